# baseline (speedup 1.0000x reference)
import jax
import jax.numpy as jnp
from jax import lax
from jax.experimental import pallas as pl
from jax.experimental.pallas import tpu as pltpu
from jax.experimental.pallas import tpu_sc as plsc

S_DIM = 1024
T_DIM = 2048
N_DATA = 100000
FEAT_DIM = 128
NCE_K = 512
NCE_T = 0.07
BSZ = 1024
EPS = 1e-07

NC = 2
NS = 16
NW = NC * NS
S_PER_W = BSZ // NW
C_PER_S = NCE_K // 128
T_STEPS = S_PER_W * C_PER_S
NBUF = 2


def _bf16_bits(u):
    return (u + jnp.int32(0x7FFF) + ((u >> jnp.int32(16)) & jnp.int32(1))) \
        >> jnp.int32(16)


def _pack_body(m1, m2, o):
    def pack(m):
        u = lax.bitcast_convert_type(m[...], jnp.int32)
        r = _bf16_bits(u)
        lo = r[:, :64] & jnp.int32(0xFFFF)
        hi = r[:, 64:] << jnp.int32(16)
        return lo | hi

    o[...] = jnp.concatenate([pack(m1), pack(m2)], axis=1)


def _pack_banks(mem1, mem2):
    blk = N_DATA // 25
    return pl.pallas_call(
        _pack_body,
        grid=(25,),
        in_specs=[pl.BlockSpec((blk, FEAT_DIM), lambda i: (i, 0))] * 2,
        out_specs=pl.BlockSpec((blk, FEAT_DIM), lambda i: (i, 0)),
        out_shape=jax.ShapeDtypeStruct((N_DATA, FEAT_DIM), jnp.int32),
    )(mem1, mem2)


def _embed_body(f_s, w_s, b_s, f_t, w_t, b_t, v1_out, v2_out):
    y1 = jnp.dot(f_s[...], w_s[...], preferred_element_type=jnp.float32,
                 precision=lax.Precision.HIGHEST) + b_s[...]
    n1 = jnp.power(jnp.sum(y1 * y1, axis=1, keepdims=True), 0.5)
    v1_out[...] = y1 / n1
    y2 = jnp.dot(f_t[...], w_t[...], preferred_element_type=jnp.float32,
                 precision=lax.Precision.HIGHEST) + b_t[...]
    n2 = jnp.power(jnp.sum(y2 * y2, axis=1, keepdims=True), 0.5)
    v2_out[...] = y2 / n2


def _embed(f_s, w_sT, b_s, f_t, w_tT, b_t):
    return pl.pallas_call(
        _embed_body,
        out_shape=(
            jax.ShapeDtypeStruct((BSZ, FEAT_DIM), jnp.float32),
            jax.ShapeDtypeStruct((BSZ, FEAT_DIM), jnp.float32),
        ),
    )(f_s, w_sT, b_s, f_t, w_tT, b_t)


def _acc16(buf, row, vv, base):
    acc = None
    for c in range(4):
        u = buf[row, pl.ds(base + 16 * c, 16)]
        lo = lax.bitcast_convert_type(u << jnp.int32(16), jnp.float32)
        hi = lax.bitcast_convert_type(u, jnp.float32)
        term = lo * vv[c] + hi * vv[c + 4]
        acc = term if acc is None else acc + term
    return acc


_GDN = lax.GatherDimensionNumbers(
    offset_dims=(), collapsed_slice_dims=(0,), start_index_map=(0,))


def _perm(x, idx):
    return lax.gather(x, idx[:, None], dimension_numbers=_GDN,
                      slice_sizes=(1,),
                      mode=lax.GatherScatterMode.PROMISE_IN_BOUNDS)


def _lane_sum(x, xori):
    for p in xori:
        x = x + _perm(x, p)
    return x


def _dots_group(get_acc, masks, xori):
    vecs = [get_acc(j) for j in range(16)]
    for s, p, m in ((1, xori[0], masks[0]), (2, xori[1], masks[1]),
                    (4, xori[2], masks[2]), (8, xori[3], masks[3])):
        nxt = []
        for i in range(0, len(vecs), 2):
            xs = vecs[i] + _perm(vecs[i], p)
            ys = vecs[i + 1] + _perm(vecs[i + 1], p)
            nxt.append(jnp.where(m, ys, xs))
        vecs = nxt
    return vecs[0]


def _sc_body(memp, negs, posi, v1, v2,
             oa_neg, oa_pos, ob_neg, ob_pos,
             nidx, vloc1, vloc2, prow, rb0, rb1, dout_a, dout_b,
             pdots_a, pdots_b, pidx, sem0, sem1):
    rbufs = [rb0, rb1]
    sems = [sem0, sem1]
    wid = lax.axis_index("s") * NC + lax.axis_index("c")
    base_c = wid * T_STEPS
    base_s = wid * S_PER_W

    lane = lax.iota(jnp.int32, 16)
    masks = [(lane & s) != 0 for s in (1, 2, 4, 8)]
    xori = [lane ^ s for s in (1, 2, 4, 8)]

    pltpu.sync_copy(negs.at[pl.ds(base_c, T_STEPS)], nidx)
    pltpu.sync_copy(v1.at[pl.ds(base_s, S_PER_W)], vloc1)
    pltpu.sync_copy(v2.at[pl.ds(base_s, S_PER_W)], vloc2)
    pltpu.sync_copy(posi.at[pl.ds(base_s, S_PER_W)], pidx)


    pltpu.async_copy(memp.at[pidx], prow, sem0).wait()

    def _pos_group(g, _):
        def acc_a(j):
            s = 16 * g + j
            vv = [vloc1[s, pl.ds(16 * r, 16)] for r in range(8)]
            return _acc16(prow, s, vv, 64)

        def acc_b(j):
            s = 16 * g + j
            vv = [vloc2[s, pl.ds(16 * r, 16)] for r in range(8)]
            return _acc16(prow, s, vv, 0)

        pdots_a[pl.ds(16 * g, 16)] = _dots_group(acc_a, masks, xori)
        pdots_b[pl.ds(16 * g, 16)] = _dots_group(acc_b, masks, xori)
        return _

    lax.fori_loop(0, S_PER_W // 16, _pos_group, None)
    pltpu.sync_copy(pdots_a, oa_pos.at[pl.ds(base_s, S_PER_W)])
    pltpu.sync_copy(pdots_b, ob_pos.at[pl.ds(base_s, S_PER_W)])

    for b in range(NBUF):
        pltpu.async_copy(memp.at[nidx.at[b]], rbufs[b], sems[b])

    def _chunk(t, buf, sem):
        s = t // C_PER_S
        vva = [vloc1[s, pl.ds(16 * r, 16)] for r in range(8)]
        vvb = [vloc2[s, pl.ds(16 * r, 16)] for r in range(8)]
        pltpu.make_async_copy(memp.at[nidx.at[t]], buf, sem).wait()

        def _k_group(g, _):
            dout_a[t, pl.ds(16 * g, 16)] = _dots_group(
                lambda j: _acc16(buf, 16 * g + j, vva, 64), masks, xori)
            dout_b[t, pl.ds(16 * g, 16)] = _dots_group(
                lambda j: _acc16(buf, 16 * g + j, vvb, 0), masks, xori)
            return _

        lax.fori_loop(0, 8, _k_group, None)

    def _t_step(i, _):
        for par in range(NBUF):
            t = NBUF * i + par
            _chunk(t, rbufs[par], sems[par])

            @pl.when(t + NBUF < T_STEPS)
            def _():
                pltpu.async_copy(
                    memp.at[nidx.at[t + NBUF]], rbufs[par], sems[par])
        return _

    lax.fori_loop(0, T_STEPS // NBUF, _t_step, None)
    pltpu.sync_copy(dout_a, oa_neg.at[pl.ds(base_c, T_STEPS)])
    pltpu.sync_copy(dout_b, ob_neg.at[pl.ds(base_c, T_STEPS)])


def _sc_dots(memp, negs, posi, v1, v2):
    mesh = plsc.VectorSubcoreMesh(core_axis_name="c", subcore_axis_name="s")
    f32 = jnp.float32
    call = pl.kernel(
        _sc_body,
        out_type=(
            jax.ShapeDtypeStruct((NW * T_STEPS, 128), f32),
            jax.ShapeDtypeStruct((BSZ,), f32),
            jax.ShapeDtypeStruct((NW * T_STEPS, 128), f32),
            jax.ShapeDtypeStruct((BSZ,), f32),
        ),
        mesh=mesh,
        scratch_types=[
            pltpu.VMEM((T_STEPS, 128), jnp.int32),
            pltpu.VMEM((S_PER_W, FEAT_DIM), f32),
            pltpu.VMEM((S_PER_W, FEAT_DIM), f32),
            pltpu.VMEM((S_PER_W, 128), jnp.int32),
            pltpu.VMEM((128, 128), jnp.int32),
            pltpu.VMEM((128, 128), jnp.int32),
            pltpu.VMEM((T_STEPS, 128), f32),
            pltpu.VMEM((T_STEPS, 128), f32),
            pltpu.VMEM((S_PER_W,), f32),
            pltpu.VMEM((S_PER_W,), f32),
            pltpu.VMEM((S_PER_W,), jnp.int32),
            pltpu.SemaphoreType.DMA,
            pltpu.SemaphoreType.DMA,
        ],
    )
    return call(memp, negs, posi, v1, v2)


def _loss_body(da_neg, da_pos, db_neg, db_pos, out):
    n_all = float(BSZ * (NCE_K + 1))
    c = float(NCE_K) * (1.0 / float(N_DATA))

    def one(neg_ref, pos_ref):
        e_neg = jnp.exp(neg_ref[...] * (1.0 / NCE_T))
        e_pos = jnp.exp(pos_ref[...] * (1.0 / NCE_T))
        z = (jnp.sum(e_neg) + jnp.sum(e_pos)) / n_all * float(N_DATA)
        p_pos = e_pos / z
        p_neg = e_neg / z
        log_d1 = jnp.log(p_pos / (p_pos + c + EPS))
        log_d0 = jnp.log(c / (p_neg + c + EPS))
        return -(jnp.sum(log_d1) + jnp.sum(log_d0)) / float(BSZ)

    out[...] = jnp.reshape(one(da_neg, da_pos) + one(db_neg, db_pos), (1, 1))


def _loss(da_neg, da_pos, db_neg, db_pos):
    return pl.pallas_call(
        _loss_body,
        out_shape=jax.ShapeDtypeStruct((1, 1), jnp.float32),
    )(da_neg, da_pos, db_neg, db_pos)


def kernel(f_s, f_t, idx, contrast_idx, W_s, b_s, W_t, b_t, memory_v1, memory_v2):
    memp = _pack_banks(memory_v1, memory_v2)
    v1, v2 = _embed(f_s, W_s.T, b_s.reshape(1, FEAT_DIM),
                    f_t, W_t.T, b_t.reshape(1, FEAT_DIM))
    negs = contrast_idx[:, 1:].reshape(NW * T_STEPS, 128)
    posi = contrast_idx[:, 0]
    oa_neg, oa_pos, ob_neg, ob_pos = _sc_dots(memp, negs, posi, v1, v2)
    loss = _loss(oa_neg, oa_pos.reshape(8, 128), ob_neg, ob_pos.reshape(8, 128))
    return loss.reshape(1)


# speedup vs baseline: 1.0193x; 1.0193x over previous
import jax
import jax.numpy as jnp
from jax import lax
from jax.experimental import pallas as pl
from jax.experimental.pallas import tpu as pltpu
from jax.experimental.pallas import tpu_sc as plsc

S_DIM = 1024
T_DIM = 2048
N_DATA = 100000
FEAT_DIM = 128
NCE_K = 512
NCE_T = 0.07
BSZ = 1024
EPS = 1e-07

NC = 2
NS = 16
NW = NC * NS
S_PER_W = BSZ // NW
C_PER_S = NCE_K // 128
T_STEPS = S_PER_W * C_PER_S
NBUF = 4


def _bf16_bits(u):
    return (u + jnp.int32(0x7FFF) + ((u >> jnp.int32(16)) & jnp.int32(1))) \
        >> jnp.int32(16)


def _pack_body(m1, m2, f_s, w_s, b_s, f_t, w_t, b_t, o, v1_out, v2_out):
    def pack(m):
        u = lax.bitcast_convert_type(m[...], jnp.int32)
        r = _bf16_bits(u)
        lo = r[:, :64] & jnp.int32(0xFFFF)
        hi = r[:, 64:] << jnp.int32(16)
        return lo | hi

    o[...] = jnp.concatenate([pack(m1), pack(m2)], axis=1)

    @pl.when(pl.program_id(0) == 0)
    def _():
        y1 = jnp.dot(f_s[...], w_s[...], preferred_element_type=jnp.float32,
                     precision=lax.Precision.HIGHEST) + b_s[...]
        n1 = jnp.power(jnp.sum(y1 * y1, axis=1, keepdims=True), 0.5)
        v1_out[...] = y1 / n1
        y2 = jnp.dot(f_t[...], w_t[...], preferred_element_type=jnp.float32,
                     precision=lax.Precision.HIGHEST) + b_t[...]
        n2 = jnp.power(jnp.sum(y2 * y2, axis=1, keepdims=True), 0.5)
        v2_out[...] = y2 / n2


def _pack_banks(mem1, mem2, f_s, w_sT, b_s, f_t, w_tT, b_t):
    blk = N_DATA // 25
    z = lambda i: (0, 0)
    return pl.pallas_call(
        _pack_body,
        grid=(25,),
        in_specs=[pl.BlockSpec((blk, FEAT_DIM), lambda i: (i, 0))] * 2 + [
            pl.BlockSpec((BSZ, S_DIM), z), pl.BlockSpec((S_DIM, FEAT_DIM), z),
            pl.BlockSpec((1, FEAT_DIM), z),
            pl.BlockSpec((BSZ, T_DIM), z), pl.BlockSpec((T_DIM, FEAT_DIM), z),
            pl.BlockSpec((1, FEAT_DIM), z),
        ],
        out_specs=(pl.BlockSpec((blk, FEAT_DIM), lambda i: (i, 0)),
                   pl.BlockSpec((BSZ, FEAT_DIM), z),
                   pl.BlockSpec((BSZ, FEAT_DIM), z)),
        out_shape=(jax.ShapeDtypeStruct((N_DATA, FEAT_DIM), jnp.int32),
                   jax.ShapeDtypeStruct((BSZ, FEAT_DIM), jnp.float32),
                   jax.ShapeDtypeStruct((BSZ, FEAT_DIM), jnp.float32)),
    )(mem1, mem2, f_s, w_sT, b_s, f_t, w_tT, b_t)


def _acc16(buf, row, vv, base):
    acc = None
    for c in range(4):
        u = buf[row, pl.ds(base + 16 * c, 16)]
        lo = lax.bitcast_convert_type(u << jnp.int32(16), jnp.float32)
        hi = lax.bitcast_convert_type(u, jnp.float32)
        term = lo * vv[c] + hi * vv[c + 4]
        acc = term if acc is None else acc + term
    return acc


_GDN = lax.GatherDimensionNumbers(
    offset_dims=(), collapsed_slice_dims=(0,), start_index_map=(0,))


def _perm(x, idx):
    return lax.gather(x, idx[:, None], dimension_numbers=_GDN,
                      slice_sizes=(1,),
                      mode=lax.GatherScatterMode.PROMISE_IN_BOUNDS)


def _lane_sum(x, xori):
    for p in xori:
        x = x + _perm(x, p)
    return x


def _dots_group(get_acc, masks, xori):
    vecs = [get_acc(j) for j in range(16)]
    for s, p, m in ((1, xori[0], masks[0]), (2, xori[1], masks[1]),
                    (4, xori[2], masks[2]), (8, xori[3], masks[3])):
        nxt = []
        for i in range(0, len(vecs), 2):
            xs = vecs[i] + _perm(vecs[i], p)
            ys = vecs[i + 1] + _perm(vecs[i + 1], p)
            nxt.append(jnp.where(m, ys, xs))
        vecs = nxt
    return vecs[0]


def _sc_body(memp, negs, posi, v1, v2,
             oa_neg, oa_pos, ob_neg, ob_pos,
             nidx, vloc1, vloc2, prow, rb0, rb1, rb2, rb3, dout_a, dout_b,
             pdots_a, pdots_b, pidx, sem0, sem1, sem2, sem3, semp):
    rbufs = [rb0, rb1, rb2, rb3]
    sems = [sem0, sem1, sem2, sem3]
    wid = lax.axis_index("s") * NC + lax.axis_index("c")
    base_c = wid * T_STEPS
    base_s = wid * S_PER_W

    lane = lax.iota(jnp.int32, 16)
    masks = [(lane & s) != 0 for s in (1, 2, 4, 8)]
    xori = [lane ^ s for s in (1, 2, 4, 8)]

    pltpu.sync_copy(negs.at[pl.ds(base_c, T_STEPS)], nidx)
    pltpu.sync_copy(v1.at[pl.ds(base_s, S_PER_W)], vloc1)
    pltpu.sync_copy(v2.at[pl.ds(base_s, S_PER_W)], vloc2)
    pltpu.sync_copy(posi.at[pl.ds(base_s, S_PER_W)], pidx)


    for b in range(NBUF):
        pltpu.async_copy(memp.at[nidx.at[b]], rbufs[b], sems[b])
    pltpu.async_copy(memp.at[pidx], prow, semp).wait()

    def _pos_group(g, _):
        def acc_a(j):
            s = 16 * g + j
            vv = [vloc1[s, pl.ds(16 * r, 16)] for r in range(8)]
            return _acc16(prow, s, vv, 64)

        def acc_b(j):
            s = 16 * g + j
            vv = [vloc2[s, pl.ds(16 * r, 16)] for r in range(8)]
            return _acc16(prow, s, vv, 0)

        pdots_a[pl.ds(16 * g, 16)] = _dots_group(acc_a, masks, xori)
        pdots_b[pl.ds(16 * g, 16)] = _dots_group(acc_b, masks, xori)
        return _

    lax.fori_loop(0, S_PER_W // 16, _pos_group, None)
    pltpu.sync_copy(pdots_a, oa_pos.at[pl.ds(base_s, S_PER_W)])
    pltpu.sync_copy(pdots_b, ob_pos.at[pl.ds(base_s, S_PER_W)])

    def _chunk(t, buf, sem):
        s = t // C_PER_S
        vva = [vloc1[s, pl.ds(16 * r, 16)] for r in range(8)]
        vvb = [vloc2[s, pl.ds(16 * r, 16)] for r in range(8)]
        pltpu.make_async_copy(memp.at[nidx.at[t]], buf, sem).wait()

        def _k_group(g, _):
            dout_a[t, pl.ds(16 * g, 16)] = _dots_group(
                lambda j: _acc16(buf, 16 * g + j, vva, 64), masks, xori)
            dout_b[t, pl.ds(16 * g, 16)] = _dots_group(
                lambda j: _acc16(buf, 16 * g + j, vvb, 0), masks, xori)
            return _

        lax.fori_loop(0, 8, _k_group, None)

    def _t_step(i, _):
        for par in range(NBUF):
            t = NBUF * i + par
            _chunk(t, rbufs[par], sems[par])

            @pl.when(t + NBUF < T_STEPS)
            def _():
                pltpu.async_copy(
                    memp.at[nidx.at[t + NBUF]], rbufs[par], sems[par])
        return _

    lax.fori_loop(0, T_STEPS // NBUF, _t_step, None)
    pltpu.sync_copy(dout_a, oa_neg.at[pl.ds(base_c, T_STEPS)])
    pltpu.sync_copy(dout_b, ob_neg.at[pl.ds(base_c, T_STEPS)])


def _sc_dots(memp, negs, posi, v1, v2):
    mesh = plsc.VectorSubcoreMesh(core_axis_name="c", subcore_axis_name="s")
    f32 = jnp.float32
    call = pl.kernel(
        _sc_body,
        out_type=(
            jax.ShapeDtypeStruct((NW * T_STEPS, 128), f32),
            jax.ShapeDtypeStruct((BSZ,), f32),
            jax.ShapeDtypeStruct((NW * T_STEPS, 128), f32),
            jax.ShapeDtypeStruct((BSZ,), f32),
        ),
        mesh=mesh,
        scratch_types=[
            pltpu.VMEM((T_STEPS, 128), jnp.int32),
            pltpu.VMEM((S_PER_W, FEAT_DIM), f32),
            pltpu.VMEM((S_PER_W, FEAT_DIM), f32),
            pltpu.VMEM((S_PER_W, 128), jnp.int32),
            pltpu.VMEM((128, 128), jnp.int32),
            pltpu.VMEM((128, 128), jnp.int32),
            pltpu.VMEM((128, 128), jnp.int32),
            pltpu.VMEM((128, 128), jnp.int32),
            pltpu.VMEM((T_STEPS, 128), f32),
            pltpu.VMEM((T_STEPS, 128), f32),
            pltpu.VMEM((S_PER_W,), f32),
            pltpu.VMEM((S_PER_W,), f32),
            pltpu.VMEM((S_PER_W,), jnp.int32),
            pltpu.SemaphoreType.DMA,
            pltpu.SemaphoreType.DMA,
            pltpu.SemaphoreType.DMA,
            pltpu.SemaphoreType.DMA,
            pltpu.SemaphoreType.DMA,
        ],
    )
    return call(memp, negs, posi, v1, v2)


def _loss_body(da_neg, da_pos, db_neg, db_pos, out):
    n_all = float(BSZ * (NCE_K + 1))
    c = float(NCE_K) * (1.0 / float(N_DATA))

    def one(neg_ref, pos_ref):
        e_neg = jnp.exp(neg_ref[...] * (1.0 / NCE_T))
        e_pos = jnp.exp(pos_ref[...] * (1.0 / NCE_T))
        z = (jnp.sum(e_neg) + jnp.sum(e_pos)) / n_all * float(N_DATA)
        p_pos = e_pos / z
        p_neg = e_neg / z
        log_d1 = jnp.log(p_pos / (p_pos + c + EPS))
        log_d0 = jnp.log(c / (p_neg + c + EPS))
        return -(jnp.sum(log_d1) + jnp.sum(log_d0)) / float(BSZ)

    out[...] = jnp.reshape(one(da_neg, da_pos) + one(db_neg, db_pos), (1, 1))


def _loss(da_neg, da_pos, db_neg, db_pos):
    return pl.pallas_call(
        _loss_body,
        out_shape=jax.ShapeDtypeStruct((1, 1), jnp.float32),
    )(da_neg, da_pos, db_neg, db_pos)


def kernel(f_s, f_t, idx, contrast_idx, W_s, b_s, W_t, b_t, memory_v1, memory_v2):
    memp, v1, v2 = _pack_banks(memory_v1, memory_v2,
                               f_s, W_s.T, b_s.reshape(1, FEAT_DIM),
                               f_t, W_t.T, b_t.reshape(1, FEAT_DIM))
    negs = contrast_idx[:, 1:].reshape(NW * T_STEPS, 128)
    posi = contrast_idx[:, 0]
    oa_neg, oa_pos, ob_neg, ob_pos = _sc_dots(memp, negs, posi, v1, v2)
    loss = _loss(oa_neg, oa_pos.reshape(8, 128), ob_neg, ob_pos.reshape(8, 128))
    return loss.reshape(1)
